# baseline (speedup 1.0000x reference)
import jax
import jax.numpy as jnp
from jax import lax
from jax.experimental import pallas as pl
from jax.experimental.pallas import tpu as pltpu
from jax.experimental.pallas import tpu_sc as plsc

N = 10000
D = 128
HEADS = 8
CH = 16
OUT = 40

R = 10112
NSC = 2
NTILE = 16
CHUNK = 96
PER_TILE = 10176
NCHUNK = PER_TILE // CHUNK
EPAD = NSC * NTILE * PER_TILE
ROWS_PER_TILE = R // NTILE

BN = 2528
NBLK = R // BN


def _tc_pre_body(x_ref, w_ref, as_ref, ad_ref, h_out, a_out, d_out, ws_out):
    h = jnp.dot(x_ref[...], w_ref[...], preferred_element_type=jnp.float32)
    a = jnp.dot(h, as_ref[...], preferred_element_type=jnp.float32)
    dd = jnp.dot(h, ad_ref[...], preferred_element_type=jnp.float32)
    e = a + dd
    e = jnp.where(e > 0, e, 0.2 * e)
    h_out[...] = h
    a_out[...] = a
    d_out[...] = dd
    ws_out[...] = jnp.exp(e)


def _tc_pre(x_pad, W, As_e, Ad_e):
    return pl.pallas_call(
        _tc_pre_body,
        grid=(NBLK,),
        in_specs=[
            pl.BlockSpec((BN, D), lambda i: (i, 0)),
            pl.BlockSpec((D, D), lambda i: (0, 0)),
            pl.BlockSpec((D, CH), lambda i: (0, 0)),
            pl.BlockSpec((D, CH), lambda i: (0, 0)),
        ],
        out_specs=[
            pl.BlockSpec((BN, D), lambda i: (i, 0)),
            pl.BlockSpec((BN, CH), lambda i: (i, 0)),
            pl.BlockSpec((BN, CH), lambda i: (i, 0)),
            pl.BlockSpec((BN, CH), lambda i: (i, 0)),
        ],
        out_shape=[
            jax.ShapeDtypeStruct((R, D), jnp.float32),
            jax.ShapeDtypeStruct((R, CH), jnp.float32),
            jax.ShapeDtypeStruct((R, CH), jnp.float32),
            jax.ShapeDtypeStruct((R, CH), jnp.float32),
        ],
    )(x_pad, W, As_e, Ad_e)


def _tc_mid_body(sa_ref, sb_ref, za_ref, zb_ref, h_ref, ws_ref, p_ref, b_ref,
                 w1_ref, as_ref, ad_ref,
                 h1_out, g1_out, a1_out, d1_out, ws1_out):
    i = pl.program_id(0)
    s = sa_ref[...] + sb_ref[...]
    z16 = za_ref[...] + zb_ref[...]
    z128 = jnp.dot(z16, p_ref[...], preferred_element_type=jnp.float32)
    w128 = jnp.dot(ws_ref[...], p_ref[...], preferred_element_type=jnp.float32)
    h0 = h_ref[...]
    out0 = (s + w128 * h0) / (z128 + w128 + 1e-16)
    h1 = out0 + b_ref[...]
    h1 = jnp.where(h1 > 0, h1, jnp.exp(h1) - 1.0)
    rows = i * BN + lax.broadcasted_iota(jnp.int32, h1.shape, 0)
    h1 = jnp.where(rows < N, h1, 0.0)
    g1 = jnp.dot(h1, w1_ref[...], preferred_element_type=jnp.float32)
    a1 = jnp.dot(g1, as_ref[...], preferred_element_type=jnp.float32)
    d1 = jnp.dot(g1, ad_ref[...], preferred_element_type=jnp.float32)
    e = a1 + d1
    e = jnp.where(e > 0, e, 0.2 * e)
    h1_out[...] = h1
    g1_out[...] = g1
    a1_out[...] = a1
    d1_out[...] = d1
    ws1_out[...] = jnp.exp(e)


def _tc_mid(S_out, Z_out, h0, ws0, P, b0_row, W1, As_e, Ad_e):
    return pl.pallas_call(
        _tc_mid_body,
        grid=(NBLK,),
        in_specs=[
            pl.BlockSpec((BN, D), lambda i: (i, 0)),
            pl.BlockSpec((BN, D), lambda i: (i + NBLK, 0)),
            pl.BlockSpec((BN, CH), lambda i: (i, 0)),
            pl.BlockSpec((BN, CH), lambda i: (i + NBLK, 0)),
            pl.BlockSpec((BN, D), lambda i: (i, 0)),
            pl.BlockSpec((BN, CH), lambda i: (i, 0)),
            pl.BlockSpec((CH, D), lambda i: (0, 0)),
            pl.BlockSpec((1, D), lambda i: (0, 0)),
            pl.BlockSpec((D, D), lambda i: (0, 0)),
            pl.BlockSpec((D, CH), lambda i: (0, 0)),
            pl.BlockSpec((D, CH), lambda i: (0, 0)),
        ],
        out_specs=[
            pl.BlockSpec((BN, D), lambda i: (i, 0)),
            pl.BlockSpec((BN, D), lambda i: (i, 0)),
            pl.BlockSpec((BN, CH), lambda i: (i, 0)),
            pl.BlockSpec((BN, CH), lambda i: (i, 0)),
            pl.BlockSpec((BN, CH), lambda i: (i, 0)),
        ],
        out_shape=[
            jax.ShapeDtypeStruct((R, D), jnp.float32),
            jax.ShapeDtypeStruct((R, D), jnp.float32),
            jax.ShapeDtypeStruct((R, CH), jnp.float32),
            jax.ShapeDtypeStruct((R, CH), jnp.float32),
            jax.ShapeDtypeStruct((R, CH), jnp.float32),
        ],
    )(S_out, S_out, Z_out, Z_out, h0, ws0, P, b0_row, W1, As_e, Ad_e)


def _tc_fin_body(sa_ref, sb_ref, za_ref, zb_ref, g_ref, h_ref, ws_ref, p_ref,
                 b_ref, wf_ref, bf_ref, out_ref):
    s = sa_ref[...] + sb_ref[...]
    z16 = za_ref[...] + zb_ref[...]
    z128 = jnp.dot(z16, p_ref[...], preferred_element_type=jnp.float32)
    w128 = jnp.dot(ws_ref[...], p_ref[...], preferred_element_type=jnp.float32)
    h1 = h_ref[...]
    h2 = (s + w128 * g_ref[...]) / (z128 + w128 + 1e-16) + b_ref[...]
    jk = jnp.maximum(h1, h2)
    logits = jnp.dot(jk, wf_ref[...], preferred_element_type=jnp.float32)
    logits = logits + bf_ref[...]
    col = lax.broadcasted_iota(jnp.int32, logits.shape, 1)
    valid = col < OUT
    masked = jnp.where(valid, logits, -1e30)
    mx = jnp.max(masked, axis=1, keepdims=True)
    ex = jnp.where(valid, jnp.exp(logits - mx), 0.0)
    lse = jnp.log(jnp.sum(ex, axis=1, keepdims=True))
    res = logits - mx - lse
    out_ref[...] = res[:, :OUT]


def _tc_fin(S_out, Z_out, g1, h1, ws1, P, b1_row, Wf_pad, bf_row):
    return pl.pallas_call(
        _tc_fin_body,
        grid=(NBLK,),
        in_specs=[
            pl.BlockSpec((BN, D), lambda i: (i, 0)),
            pl.BlockSpec((BN, D), lambda i: (i + NBLK, 0)),
            pl.BlockSpec((BN, CH), lambda i: (i, 0)),
            pl.BlockSpec((BN, CH), lambda i: (i + NBLK, 0)),
            pl.BlockSpec((BN, D), lambda i: (i, 0)),
            pl.BlockSpec((BN, D), lambda i: (i, 0)),
            pl.BlockSpec((BN, CH), lambda i: (i, 0)),
            pl.BlockSpec((CH, D), lambda i: (0, 0)),
            pl.BlockSpec((1, D), lambda i: (0, 0)),
            pl.BlockSpec((D, D), lambda i: (0, 0)),
            pl.BlockSpec((1, D), lambda i: (0, 0)),
        ],
        out_specs=pl.BlockSpec((BN, OUT), lambda i: (i, 0)),
        out_shape=jax.ShapeDtypeStruct((N, OUT), jnp.float32),
    )(S_out, S_out, Z_out, Z_out, g1, h1, ws1, P, b1_row, Wf_pad, bf_row)


def _sc_edge_body(src_hbm, dst_hbm, a_hbm, d_hbm, h_hbm, zs_hbm, zz_hbm,
                  s_out, z_out,
                  src_v, dst_v, a_v, d_v, h_v, w_v, s_sh, z_sh,
                  isem, gsem, ssem):
    c = lax.axis_index("c")
    s = lax.axis_index("s")
    wid = s * NSC + c
    r0 = s * ROWS_PER_TILE
    rows = pl.ds(r0, ROWS_PER_TILE)
    base = wid * PER_TILE

    def start_idx(j, sp, dp):
        pltpu.async_copy(src_hbm.at[pl.ds(base + j * CHUNK, CHUNK)],
                         src_v.at[sp], isem)
        pltpu.async_copy(dst_hbm.at[pl.ds(base + j * CHUNK, CHUNK)],
                         dst_v.at[dp], isem)

    def wait_idx(sp, dp):
        pltpu.make_async_copy(src_hbm.at[pl.ds(base, CHUNK)],
                              src_v.at[sp], isem).wait()
        pltpu.make_async_copy(dst_hbm.at[pl.ds(base, CHUNK)],
                              dst_v.at[dp], isem).wait()

    def start_gathers(sp, dp, p):
        pltpu.async_copy(a_hbm.at[src_v.at[sp]], a_v.at[p], gsem)
        pltpu.async_copy(d_hbm.at[dst_v.at[dp]], d_v.at[p], gsem)
        pltpu.async_copy(h_hbm.at[src_v.at[sp]], h_v.at[p], gsem)

    def wait_gathers(sp, dp, p):
        pltpu.make_async_copy(a_hbm.at[src_v.at[sp]], a_v.at[p], gsem).wait()
        pltpu.make_async_copy(d_hbm.at[dst_v.at[dp]], d_v.at[p], gsem).wait()
        pltpu.make_async_copy(h_hbm.at[src_v.at[sp]], h_v.at[p], gsem).wait()

    def drain_scatters(dp, p):
        pltpu.make_async_copy(h_v.at[p], s_sh.at[dst_v.at[dp]],
                              ssem.at[p]).wait()
        pltpu.make_async_copy(w_v.at[p], z_sh.at[dst_v.at[dp]],
                              ssem.at[p]).wait()

    zc = pltpu.async_copy(zs_hbm, s_sh.at[rows, :], isem)
    zc2 = pltpu.async_copy(zz_hbm, z_sh.at[rows, :], isem)
    start_idx(0, 0, 0)
    zc.wait()
    zc2.wait()
    wait_idx(0, 0)
    plsc.subcore_barrier()
    start_gathers(0, 0, 0)
    start_idx(1, 1, 1)

    def chunk_body(j, carry):
        p2 = lax.rem(j, 2)
        p4 = lax.rem(j, 4)
        wait_gathers(p2, p4, p2)

        @pl.when(j >= 1)
        def _():
            drain_scatters(lax.rem(j + 3, 4), 1 - p2)

        @pl.when(j + 1 < NCHUNK)
        def _():
            wait_idx(1 - p2, lax.rem(j + 1, 4))
            start_gathers(1 - p2, lax.rem(j + 1, 4), 1 - p2)

        @pl.when(j + 2 < NCHUNK)
        def _():
            start_idx(j + 2, p2, lax.rem(j + 2, 4))

        @plsc.parallel_loop(0, CHUNK, unroll=8)
        def row_body(i):
            e16 = a_v[p2, i, :] + d_v[p2, i, :]
            e16 = jnp.where(e16 > 0, e16, e16 * 0.2)
            w16 = jnp.exp(e16)
            w_v[p2, i, :] = w16
            for hd in range(HEADS):
                ws = w16[hd]
                h_v[p2, i, pl.ds(hd * CH, CH)] = (
                    ws * h_v[p2, i, pl.ds(hd * CH, CH)])

        pltpu.async_copy(h_v.at[p2], s_sh.at[dst_v.at[p4]], ssem.at[p2],
                         add=True)
        pltpu.async_copy(w_v.at[p2], z_sh.at[dst_v.at[p4]], ssem.at[p2],
                         add=True)
        return carry

    lax.fori_loop(0, NCHUNK, chunk_body, 0)
    drain_scatters(lax.rem(NCHUNK - 1, 4), (NCHUNK - 1) % 2)
    plsc.subcore_barrier()

    pltpu.sync_copy(s_sh.at[rows, :],
                    s_out.at[pl.ds(c * R + r0, ROWS_PER_TILE), :])
    pltpu.sync_copy(z_sh.at[rows, :],
                    z_out.at[pl.ds(c * R + r0, ROWS_PER_TILE), :])


def _sc_edge_pass(srcp, dstp, A, Dt, H, zeroS, zeroZ):
    mesh = plsc.VectorSubcoreMesh(core_axis_name="c", subcore_axis_name="s")
    fn = pl.kernel(
        _sc_edge_body,
        mesh=mesh,
        compiler_params=pltpu.CompilerParams(use_tc_tiling_on_sc=False),
        out_type=[
            jax.ShapeDtypeStruct((NSC * R, D), jnp.float32),
            jax.ShapeDtypeStruct((NSC * R, CH), jnp.float32),
        ],
        scratch_types=[
            pltpu.VMEM((2, CHUNK), jnp.int32),
            pltpu.VMEM((4, CHUNK), jnp.int32),
            pltpu.VMEM((2, CHUNK, CH), jnp.float32),
            pltpu.VMEM((2, CHUNK, CH), jnp.float32),
            pltpu.VMEM((2, CHUNK, D), jnp.float32),
            pltpu.VMEM((2, CHUNK, CH), jnp.float32),
            pltpu.VMEM_SHARED((R, D), jnp.float32),
            pltpu.VMEM_SHARED((R, CH), jnp.float32),
            pltpu.SemaphoreType.DMA,
            pltpu.SemaphoreType.DMA,
            pltpu.SemaphoreType.DMA((2,)),
        ],
    )
    return fn(srcp, dstp, A, Dt, H, zeroS, zeroZ)


def _expand_att(a):
    af = a.reshape(-1)
    heads = jnp.arange(D, dtype=jnp.int32) // CH
    cols = jnp.arange(CH, dtype=jnp.int32)
    m = (cols[None, :] % HEADS) == heads[:, None]
    return jnp.where(m, af[:, None], 0.0).astype(jnp.float32)


def kernel(x, edge_index, W0, a_s0, a_d0, b0, W1, a_s1, a_d1, b1, Wf, bf):
    f32 = jnp.float32
    pad_e = EPAD - edge_index.shape[1]
    pad_ids = N + (jnp.arange(pad_e, dtype=jnp.int32) % (R - N))
    srcp = jnp.concatenate([edge_index[0], pad_ids])
    dstp = jnp.concatenate([edge_index[1], pad_ids])

    As0 = _expand_att(a_s0)
    Ad0 = _expand_att(a_d0)
    As1 = _expand_att(a_s1)
    Ad1 = _expand_att(a_d1)
    P = ((jnp.arange(CH)[:, None] == (jnp.arange(D)[None, :] // CH))
         .astype(f32))
    b0_row = b0.reshape(1, D).astype(f32)
    b1_row = b1.reshape(1, D).astype(f32)
    Wf_pad = jnp.zeros((D, D), f32).at[:, :OUT].set(Wf)
    bf_row = jnp.zeros((1, D), f32).at[0, :OUT].set(bf)
    zeroS = jnp.zeros((ROWS_PER_TILE, D), f32)
    zeroZ = jnp.zeros((ROWS_PER_TILE, CH), f32)

    h0, a0, d0, ws0 = _tc_pre(x, W0, As0, Ad0)
    S0, Z0 = _sc_edge_pass(srcp, dstp, a0, d0, h0, zeroS, zeroZ)
    h1, g1, a1, d1, ws1 = _tc_mid(S0, Z0, h0, ws0, P, b0_row, W1, As1, Ad1)
    S1, Z1 = _sc_edge_pass(srcp, dstp, a1, d1, g1, zeroS, zeroZ)
    return _tc_fin(S1, Z1, g1, h1, ws1, P, b1_row, Wf_pad, bf_row)


# speedup vs baseline: 1.0586x; 1.0586x over previous
import jax
import jax.numpy as jnp
from jax import lax
from jax.experimental import pallas as pl
from jax.experimental.pallas import tpu as pltpu
from jax.experimental.pallas import tpu_sc as plsc

N = 10000
D = 128
HEADS = 8
CH = 16
OUT = 40

R = 10112
SW = D + CH
NSC = 2
NTILE = 16
CHUNK = 64
PER_TILE = 10112
NCHUNK = PER_TILE // CHUNK
EPAD = NSC * NTILE * PER_TILE
ROWS_PER_TILE = R // NTILE

BN = 2528
NBLK = R // BN


def _tc_pre_body(x_ref, w_ref, as_ref, ad_ref, h_out, a_out, d_out, ws_out):
    h = jnp.dot(x_ref[...], w_ref[...], preferred_element_type=jnp.float32)
    a = jnp.dot(h, as_ref[...], preferred_element_type=jnp.float32)
    dd = jnp.dot(h, ad_ref[...], preferred_element_type=jnp.float32)
    e = a + dd
    e = jnp.where(e > 0, e, 0.2 * e)
    h_out[...] = h
    a_out[...] = a
    d_out[...] = dd
    ws_out[...] = jnp.exp(e)


def _tc_pre(x, W, As_e, Ad_e):
    return pl.pallas_call(
        _tc_pre_body,
        grid=(NBLK,),
        in_specs=[
            pl.BlockSpec((BN, D), lambda i: (i, 0)),
            pl.BlockSpec((D, D), lambda i: (0, 0)),
            pl.BlockSpec((D, CH), lambda i: (0, 0)),
            pl.BlockSpec((D, CH), lambda i: (0, 0)),
        ],
        out_specs=[
            pl.BlockSpec((BN, D), lambda i: (i, 0)),
            pl.BlockSpec((BN, CH), lambda i: (i, 0)),
            pl.BlockSpec((BN, CH), lambda i: (i, 0)),
            pl.BlockSpec((BN, CH), lambda i: (i, 0)),
        ],
        out_shape=[
            jax.ShapeDtypeStruct((R, D), jnp.float32),
            jax.ShapeDtypeStruct((R, CH), jnp.float32),
            jax.ShapeDtypeStruct((R, CH), jnp.float32),
            jax.ShapeDtypeStruct((R, CH), jnp.float32),
        ],
    )(x, W, As_e, Ad_e)


def _tc_mid_body(sa_ref, sb_ref, h_ref, ws_ref, p_ref, b_ref,
                 w1_ref, as_ref, ad_ref,
                 h1_out, g1_out, a1_out, d1_out, ws1_out):
    i = pl.program_id(0)
    sfull = sa_ref[...] + sb_ref[...]
    s = sfull[:, :D]
    z16 = sfull[:, D:]
    z128 = jnp.dot(z16, p_ref[...], preferred_element_type=jnp.float32)
    w128 = jnp.dot(ws_ref[...], p_ref[...], preferred_element_type=jnp.float32)
    h0 = h_ref[...]
    out0 = (s + w128 * h0) / (z128 + w128 + 1e-16)
    h1 = out0 + b_ref[...]
    h1 = jnp.where(h1 > 0, h1, jnp.exp(h1) - 1.0)
    rows = i * BN + lax.broadcasted_iota(jnp.int32, h1.shape, 0)
    h1 = jnp.where(rows < N, h1, 0.0)
    g1 = jnp.dot(h1, w1_ref[...], preferred_element_type=jnp.float32)
    a1 = jnp.dot(g1, as_ref[...], preferred_element_type=jnp.float32)
    d1 = jnp.dot(g1, ad_ref[...], preferred_element_type=jnp.float32)
    e = a1 + d1
    e = jnp.where(e > 0, e, 0.2 * e)
    h1_out[...] = h1
    g1_out[...] = g1
    a1_out[...] = a1
    d1_out[...] = d1
    ws1_out[...] = jnp.exp(e)


def _tc_mid(S_out, h0, ws0, P, b0_row, W1, As_e, Ad_e):
    return pl.pallas_call(
        _tc_mid_body,
        grid=(NBLK,),
        in_specs=[
            pl.BlockSpec((BN, SW), lambda i: (i, 0)),
            pl.BlockSpec((BN, SW), lambda i: (i + NBLK, 0)),
            pl.BlockSpec((BN, D), lambda i: (i, 0)),
            pl.BlockSpec((BN, CH), lambda i: (i, 0)),
            pl.BlockSpec((CH, D), lambda i: (0, 0)),
            pl.BlockSpec((1, D), lambda i: (0, 0)),
            pl.BlockSpec((D, D), lambda i: (0, 0)),
            pl.BlockSpec((D, CH), lambda i: (0, 0)),
            pl.BlockSpec((D, CH), lambda i: (0, 0)),
        ],
        out_specs=[
            pl.BlockSpec((BN, D), lambda i: (i, 0)),
            pl.BlockSpec((BN, D), lambda i: (i, 0)),
            pl.BlockSpec((BN, CH), lambda i: (i, 0)),
            pl.BlockSpec((BN, CH), lambda i: (i, 0)),
            pl.BlockSpec((BN, CH), lambda i: (i, 0)),
        ],
        out_shape=[
            jax.ShapeDtypeStruct((R, D), jnp.float32),
            jax.ShapeDtypeStruct((R, D), jnp.float32),
            jax.ShapeDtypeStruct((R, CH), jnp.float32),
            jax.ShapeDtypeStruct((R, CH), jnp.float32),
            jax.ShapeDtypeStruct((R, CH), jnp.float32),
        ],
    )(S_out, S_out, h0, ws0, P, b0_row, W1, As_e, Ad_e)


def _tc_fin_body(sa_ref, sb_ref, g_ref, h_ref, ws_ref, p_ref,
                 b_ref, wf_ref, bf_ref, out_ref):
    sfull = sa_ref[...] + sb_ref[...]
    s = sfull[:, :D]
    z16 = sfull[:, D:]
    z128 = jnp.dot(z16, p_ref[...], preferred_element_type=jnp.float32)
    w128 = jnp.dot(ws_ref[...], p_ref[...], preferred_element_type=jnp.float32)
    h1 = h_ref[...]
    h2 = (s + w128 * g_ref[...]) / (z128 + w128 + 1e-16) + b_ref[...]
    jk = jnp.maximum(h1, h2)
    logits = jnp.dot(jk, wf_ref[...], preferred_element_type=jnp.float32)
    logits = logits + bf_ref[...]
    col = lax.broadcasted_iota(jnp.int32, logits.shape, 1)
    valid = col < OUT
    masked = jnp.where(valid, logits, -1e30)
    mx = jnp.max(masked, axis=1, keepdims=True)
    ex = jnp.where(valid, jnp.exp(logits - mx), 0.0)
    lse = jnp.log(jnp.sum(ex, axis=1, keepdims=True))
    res = logits - mx - lse
    out_ref[...] = res[:, :OUT]


def _tc_fin(S_out, g1, h1, ws1, P, b1_row, Wf_pad, bf_row):
    return pl.pallas_call(
        _tc_fin_body,
        grid=(NBLK,),
        in_specs=[
            pl.BlockSpec((BN, SW), lambda i: (i, 0)),
            pl.BlockSpec((BN, SW), lambda i: (i + NBLK, 0)),
            pl.BlockSpec((BN, D), lambda i: (i, 0)),
            pl.BlockSpec((BN, D), lambda i: (i, 0)),
            pl.BlockSpec((BN, CH), lambda i: (i, 0)),
            pl.BlockSpec((CH, D), lambda i: (0, 0)),
            pl.BlockSpec((1, D), lambda i: (0, 0)),
            pl.BlockSpec((D, D), lambda i: (0, 0)),
            pl.BlockSpec((1, D), lambda i: (0, 0)),
        ],
        out_specs=pl.BlockSpec((BN, OUT), lambda i: (i, 0)),
        out_shape=jax.ShapeDtypeStruct((N, OUT), jnp.float32),
    )(S_out, S_out, g1, h1, ws1, P, b1_row, Wf_pad, bf_row)


def _sc_edge_body(src_hbm, dst_hbm, a_hbm, d_hbm, h_hbm, zs_hbm,
                  s_out,
                  src_v, dst_v, a_v, d_v, h_v, msg_v, s_sh,
                  isem, gsem, ssem):
    c = lax.axis_index("c")
    s = lax.axis_index("s")
    wid = s * NSC + c
    r0 = s * ROWS_PER_TILE
    rows = pl.ds(r0, ROWS_PER_TILE)
    base = wid * PER_TILE

    def start_idx(j, sp, dp):
        pltpu.async_copy(src_hbm.at[pl.ds(base + j * CHUNK, CHUNK)],
                         src_v.at[sp], isem)
        pltpu.async_copy(dst_hbm.at[pl.ds(base + j * CHUNK, CHUNK)],
                         dst_v.at[dp], isem)

    def wait_idx(sp, dp):
        pltpu.make_async_copy(src_hbm.at[pl.ds(base, CHUNK)],
                              src_v.at[sp], isem).wait()
        pltpu.make_async_copy(dst_hbm.at[pl.ds(base, CHUNK)],
                              dst_v.at[dp], isem).wait()

    def start_gathers(sp, dp, p):
        pltpu.async_copy(a_hbm.at[src_v.at[sp]], a_v.at[p], gsem)
        pltpu.async_copy(d_hbm.at[dst_v.at[dp]], d_v.at[p], gsem)
        pltpu.async_copy(h_hbm.at[src_v.at[sp]], h_v.at[p], gsem)

    def wait_gathers(sp, dp, p):
        pltpu.make_async_copy(a_hbm.at[src_v.at[sp]], a_v.at[p], gsem).wait()
        pltpu.make_async_copy(d_hbm.at[dst_v.at[dp]], d_v.at[p], gsem).wait()
        pltpu.make_async_copy(h_hbm.at[src_v.at[sp]], h_v.at[p], gsem).wait()

    zc = pltpu.async_copy(zs_hbm, s_sh.at[rows, :], isem)
    start_idx(0, 0, 0)
    zc.wait()
    wait_idx(0, 0)
    plsc.subcore_barrier()
    start_gathers(0, 0, 0)
    start_idx(1, 1, 1)

    def chunk_body(j, carry):
        p2 = lax.rem(j, 2)
        p4 = lax.rem(j, 4)
        @pl.when(j >= 2)
        def _():
            pltpu.make_async_copy(msg_v.at[p2], s_sh.at[dst_v.at[p4]],
                                  ssem.at[p2]).wait()

        wait_gathers(p2, p4, p2)

        @pl.when(j + 1 < NCHUNK)
        def _():
            wait_idx(1 - p2, lax.rem(j + 1, 4))
            start_gathers(1 - p2, lax.rem(j + 1, 4), 1 - p2)

        @pl.when(j + 2 < NCHUNK)
        def _():
            start_idx(j + 2, p2, lax.rem(j + 2, 4))

        @plsc.parallel_loop(0, CHUNK, unroll=16)
        def row_body(i):
            e16 = a_v[p2, i, :] + d_v[p2, i, :]
            e16 = jnp.where(e16 > 0, e16, e16 * 0.2)
            w16 = jnp.exp(e16)
            msg_v[p2, i, pl.ds(D, CH)] = w16
            for hd in range(HEADS):
                ws = w16[hd]
                msg_v[p2, i, pl.ds(hd * CH, CH)] = (
                    ws * h_v[p2, i, pl.ds(hd * CH, CH)])

        pltpu.async_copy(msg_v.at[p2], s_sh.at[dst_v.at[p4]], ssem.at[p2],
                         add=True)
        return carry

    lax.fori_loop(0, NCHUNK, chunk_body, 0)
    pltpu.make_async_copy(msg_v.at[0], s_sh.at[dst_v.at[0]], ssem.at[0]).wait()
    pltpu.make_async_copy(msg_v.at[1], s_sh.at[dst_v.at[1]], ssem.at[1]).wait()
    plsc.subcore_barrier()

    pltpu.sync_copy(s_sh.at[rows, :],
                    s_out.at[pl.ds(c * R + r0, ROWS_PER_TILE), :])


def _sc_edge_pass(srcp, dstp, A, Dt, H, zeroS):
    mesh = plsc.VectorSubcoreMesh(core_axis_name="c", subcore_axis_name="s")
    fn = pl.kernel(
        _sc_edge_body,
        mesh=mesh,
        compiler_params=pltpu.CompilerParams(use_tc_tiling_on_sc=False),
        out_type=jax.ShapeDtypeStruct((NSC * R, SW), jnp.float32),
        scratch_types=[
            pltpu.VMEM((2, CHUNK), jnp.int32),
            pltpu.VMEM((4, CHUNK), jnp.int32),
            pltpu.VMEM((2, CHUNK, CH), jnp.float32),
            pltpu.VMEM((2, CHUNK, CH), jnp.float32),
            pltpu.VMEM((2, CHUNK, D), jnp.float32),
            pltpu.VMEM((2, CHUNK, SW), jnp.float32),
            pltpu.VMEM_SHARED((R, SW), jnp.float32),
            pltpu.SemaphoreType.DMA,
            pltpu.SemaphoreType.DMA,
            pltpu.SemaphoreType.DMA((2,)),
        ],
    )
    return fn(srcp, dstp, A, Dt, H, zeroS)


def _expand_att(a):
    af = a.reshape(-1)
    heads = jnp.arange(D, dtype=jnp.int32) // CH
    cols = jnp.arange(CH, dtype=jnp.int32)
    m = (cols[None, :] % HEADS) == heads[:, None]
    return jnp.where(m, af[:, None], 0.0).astype(jnp.float32)


def kernel(x, edge_index, W0, a_s0, a_d0, b0, W1, a_s1, a_d1, b1, Wf, bf):
    f32 = jnp.float32
    pad_e = EPAD - edge_index.shape[1]
    pad_ids = N + (jnp.arange(pad_e, dtype=jnp.int32) % (R - N))
    srcp = jnp.concatenate([edge_index[0], pad_ids])
    dstp = jnp.concatenate([edge_index[1], pad_ids])

    As0 = _expand_att(a_s0)
    Ad0 = _expand_att(a_d0)
    As1 = _expand_att(a_s1)
    Ad1 = _expand_att(a_d1)
    P = ((jnp.arange(CH)[:, None] == (jnp.arange(D)[None, :] // CH))
         .astype(f32))
    b0_row = b0.reshape(1, D).astype(f32)
    b1_row = b1.reshape(1, D).astype(f32)
    Wf_pad = jnp.zeros((D, D), f32).at[:, :OUT].set(Wf)
    bf_row = jnp.zeros((1, D), f32).at[0, :OUT].set(bf)
    zeroS = jnp.zeros((ROWS_PER_TILE, SW), f32)

    h0, a0, d0, ws0 = _tc_pre(x, W0, As0, Ad0)
    S0 = _sc_edge_pass(srcp, dstp, a0, d0, h0, zeroS)
    h1, g1, a1, d1, ws1 = _tc_mid(S0, h0, ws0, P, b0_row, W1, As1, Ad1)
    S1 = _sc_edge_pass(srcp, dstp, a1, d1, g1, zeroS)
    return _tc_fin(S1, g1, h1, ws1, P, b1_row, Wf_pad, bf_row)


# speedup vs baseline: 1.0697x; 1.0105x over previous
import jax
import jax.numpy as jnp
from jax import lax
from jax.experimental import pallas as pl
from jax.experimental.pallas import tpu as pltpu
from jax.experimental.pallas import tpu_sc as plsc

N = 10000
D = 128
HEADS = 8
CH = 16
OUT = 40

R = 10112
SW = D + CH
NSC = 2
NTILE = 16
CHUNK = 64
PER_TILE = 10112
NCHUNK = PER_TILE // CHUNK
EPAD = NSC * NTILE * PER_TILE
ROWS_PER_TILE = R // NTILE

BN = 2528
NBLK = R // BN


def _tc_pre_body(x_ref, w_ref, as_ref, ad_ref, h_out, a_out, d_out, ws_out):
    h = jnp.dot(x_ref[...], w_ref[...], preferred_element_type=jnp.float32)
    a = jnp.dot(h, as_ref[...], preferred_element_type=jnp.float32)
    dd = jnp.dot(h, ad_ref[...], preferred_element_type=jnp.float32)
    e = a + dd
    e = jnp.where(e > 0, e, 0.2 * e)
    h_out[...] = h
    a_out[...] = a
    d_out[...] = dd
    ws_out[...] = jnp.exp(e)


def _tc_pre(x, W, As_e, Ad_e):
    return pl.pallas_call(
        _tc_pre_body,
        grid=(NBLK,),
        in_specs=[
            pl.BlockSpec((BN, D), lambda i: (i, 0)),
            pl.BlockSpec((D, D), lambda i: (0, 0)),
            pl.BlockSpec((D, CH), lambda i: (0, 0)),
            pl.BlockSpec((D, CH), lambda i: (0, 0)),
        ],
        out_specs=[
            pl.BlockSpec((BN, D), lambda i: (i, 0)),
            pl.BlockSpec((BN, CH), lambda i: (i, 0)),
            pl.BlockSpec((BN, CH), lambda i: (i, 0)),
            pl.BlockSpec((BN, CH), lambda i: (i, 0)),
        ],
        out_shape=[
            jax.ShapeDtypeStruct((R, D), jnp.float32),
            jax.ShapeDtypeStruct((R, CH), jnp.float32),
            jax.ShapeDtypeStruct((R, CH), jnp.float32),
            jax.ShapeDtypeStruct((R, CH), jnp.float32),
        ],
    )(x, W, As_e, Ad_e)


def _tc_mid_body(sa_ref, sb_ref, h_ref, ws_ref, p_ref, b_ref,
                 w1_ref, as_ref, ad_ref,
                 h1_out, g1_out, a1_out, d1_out, ws1_out):
    i = pl.program_id(0)
    sfull = sa_ref[...] + sb_ref[...]
    s = sfull[:, :D]
    z16 = sfull[:, D:]
    z128 = jnp.dot(z16, p_ref[...], preferred_element_type=jnp.float32)
    w128 = jnp.dot(ws_ref[...], p_ref[...], preferred_element_type=jnp.float32)
    h0 = h_ref[...]
    out0 = (s + w128 * h0) / (z128 + w128 + 1e-16)
    h1 = out0 + b_ref[...]
    h1 = jnp.where(h1 > 0, h1, jnp.exp(h1) - 1.0)
    rows = i * BN + lax.broadcasted_iota(jnp.int32, h1.shape, 0)
    h1 = jnp.where(rows < N, h1, 0.0)
    g1 = jnp.dot(h1, w1_ref[...], preferred_element_type=jnp.float32)
    a1 = jnp.dot(g1, as_ref[...], preferred_element_type=jnp.float32)
    d1 = jnp.dot(g1, ad_ref[...], preferred_element_type=jnp.float32)
    e = a1 + d1
    e = jnp.where(e > 0, e, 0.2 * e)
    h1_out[...] = h1
    g1_out[...] = g1
    a1_out[...] = a1
    d1_out[...] = d1
    ws1_out[...] = jnp.exp(e)


def _tc_mid(S_out, h0, ws0, P, b0_row, W1, As_e, Ad_e):
    return pl.pallas_call(
        _tc_mid_body,
        grid=(NBLK,),
        in_specs=[
            pl.BlockSpec((BN, SW), lambda i: (i, 0)),
            pl.BlockSpec((BN, SW), lambda i: (i + NBLK, 0)),
            pl.BlockSpec((BN, D), lambda i: (i, 0)),
            pl.BlockSpec((BN, CH), lambda i: (i, 0)),
            pl.BlockSpec((CH, D), lambda i: (0, 0)),
            pl.BlockSpec((1, D), lambda i: (0, 0)),
            pl.BlockSpec((D, D), lambda i: (0, 0)),
            pl.BlockSpec((D, CH), lambda i: (0, 0)),
            pl.BlockSpec((D, CH), lambda i: (0, 0)),
        ],
        out_specs=[
            pl.BlockSpec((BN, D), lambda i: (i, 0)),
            pl.BlockSpec((BN, D), lambda i: (i, 0)),
            pl.BlockSpec((BN, CH), lambda i: (i, 0)),
            pl.BlockSpec((BN, CH), lambda i: (i, 0)),
            pl.BlockSpec((BN, CH), lambda i: (i, 0)),
        ],
        out_shape=[
            jax.ShapeDtypeStruct((R, D), jnp.float32),
            jax.ShapeDtypeStruct((R, D), jnp.float32),
            jax.ShapeDtypeStruct((R, CH), jnp.float32),
            jax.ShapeDtypeStruct((R, CH), jnp.float32),
            jax.ShapeDtypeStruct((R, CH), jnp.float32),
        ],
    )(S_out, S_out, h0, ws0, P, b0_row, W1, As_e, Ad_e)


def _tc_fin_body(sa_ref, sb_ref, g_ref, h_ref, ws_ref, p_ref,
                 b_ref, wf_ref, bf_ref, out_ref):
    sfull = sa_ref[...] + sb_ref[...]
    s = sfull[:, :D]
    z16 = sfull[:, D:]
    z128 = jnp.dot(z16, p_ref[...], preferred_element_type=jnp.float32)
    w128 = jnp.dot(ws_ref[...], p_ref[...], preferred_element_type=jnp.float32)
    h1 = h_ref[...]
    h2 = (s + w128 * g_ref[...]) / (z128 + w128 + 1e-16) + b_ref[...]
    jk = jnp.maximum(h1, h2)
    logits = jnp.dot(jk, wf_ref[...], preferred_element_type=jnp.float32)
    logits = logits + bf_ref[...]
    col = lax.broadcasted_iota(jnp.int32, logits.shape, 1)
    valid = col < OUT
    masked = jnp.where(valid, logits, -1e30)
    mx = jnp.max(masked, axis=1, keepdims=True)
    ex = jnp.where(valid, jnp.exp(logits - mx), 0.0)
    lse = jnp.log(jnp.sum(ex, axis=1, keepdims=True))
    res = logits - mx - lse
    out_ref[...] = res[:, :OUT]


def _tc_fin(S_out, g1, h1, ws1, P, b1_row, Wf_pad, bf_row):
    return pl.pallas_call(
        _tc_fin_body,
        grid=(NBLK,),
        in_specs=[
            pl.BlockSpec((BN, SW), lambda i: (i, 0)),
            pl.BlockSpec((BN, SW), lambda i: (i + NBLK, 0)),
            pl.BlockSpec((BN, D), lambda i: (i, 0)),
            pl.BlockSpec((BN, D), lambda i: (i, 0)),
            pl.BlockSpec((BN, CH), lambda i: (i, 0)),
            pl.BlockSpec((CH, D), lambda i: (0, 0)),
            pl.BlockSpec((1, D), lambda i: (0, 0)),
            pl.BlockSpec((D, D), lambda i: (0, 0)),
            pl.BlockSpec((1, D), lambda i: (0, 0)),
        ],
        out_specs=pl.BlockSpec((BN, OUT), lambda i: (i, 0)),
        out_shape=jax.ShapeDtypeStruct((N, OUT), jnp.float32),
    )(S_out, S_out, g1, h1, ws1, P, b1_row, Wf_pad, bf_row)


def _sc_edge_body(edg_hbm, a_hbm, d_hbm, h_hbm, zs_hbm,
                  s_out,
                  idx_v, a_v, d_v, h_v, msg_v, s_sh,
                  isem, gsem, ssem):
    c = lax.axis_index("c")
    s = lax.axis_index("s")
    wid = s * NSC + c
    r0 = s * ROWS_PER_TILE
    rows = pl.ds(r0, ROWS_PER_TILE)
    base = wid * PER_TILE

    def start_idx(j, dp):
        pltpu.async_copy(edg_hbm.at[:, pl.ds(base + j * CHUNK, CHUNK)],
                         idx_v.at[dp], isem)

    def wait_idx(dp):
        pltpu.make_async_copy(edg_hbm.at[:, pl.ds(base, CHUNK)],
                              idx_v.at[dp], isem).wait()

    def start_gathers(dp, p):
        pltpu.async_copy(a_hbm.at[idx_v.at[dp, 0]], a_v.at[p], gsem)
        pltpu.async_copy(d_hbm.at[idx_v.at[dp, 1]], d_v.at[p], gsem)
        pltpu.async_copy(h_hbm.at[idx_v.at[dp, 0]], h_v.at[p], gsem)

    def wait_gathers(dp, p):
        pltpu.make_async_copy(a_hbm.at[idx_v.at[dp, 0]], a_v.at[p], gsem).wait()
        pltpu.make_async_copy(d_hbm.at[idx_v.at[dp, 1]], d_v.at[p], gsem).wait()
        pltpu.make_async_copy(h_hbm.at[idx_v.at[dp, 0]], h_v.at[p], gsem).wait()

    zc = pltpu.async_copy(zs_hbm, s_sh.at[rows, :], isem)
    start_idx(0, 0)
    zc.wait()
    wait_idx(0)
    plsc.subcore_barrier()
    start_gathers(0, 0)
    start_idx(1, 1)

    def chunk_body(j, carry):
        p2 = lax.rem(j, 2)
        p4 = lax.rem(j, 4)
        @pl.when(j >= 2)
        def _():
            pltpu.make_async_copy(msg_v.at[p2], s_sh.at[idx_v.at[p4, 1]],
                                  ssem.at[p2]).wait()

        wait_gathers(p4, p2)

        @pl.when(j + 1 < NCHUNK)
        def _():
            wait_idx(lax.rem(j + 1, 4))
            start_gathers(lax.rem(j + 1, 4), 1 - p2)

        @pl.when(j + 2 < NCHUNK)
        def _():
            start_idx(j + 2, lax.rem(j + 2, 4))

        @plsc.parallel_loop(0, CHUNK, unroll=16)
        def row_body(i):
            e16 = a_v[p2, i, :] + d_v[p2, i, :]
            e16 = jnp.where(e16 > 0, e16, e16 * 0.2)
            w16 = jnp.exp(e16)
            msg_v[p2, i, pl.ds(D, CH)] = w16
            for hd in range(HEADS):
                ws = w16[hd]
                msg_v[p2, i, pl.ds(hd * CH, CH)] = (
                    ws * h_v[p2, i, pl.ds(hd * CH, CH)])

        pltpu.async_copy(msg_v.at[p2], s_sh.at[idx_v.at[p4, 1]], ssem.at[p2],
                         add=True)
        return carry

    lax.fori_loop(0, NCHUNK, chunk_body, 0)
    pltpu.make_async_copy(msg_v.at[0], s_sh.at[idx_v.at[0, 1]], ssem.at[0]).wait()
    pltpu.make_async_copy(msg_v.at[1], s_sh.at[idx_v.at[1, 1]], ssem.at[1]).wait()
    plsc.subcore_barrier()

    pltpu.sync_copy(s_sh.at[rows, :],
                    s_out.at[pl.ds(c * R + r0, ROWS_PER_TILE), :])


def _sc_edge_pass(edges, A, Dt, H, zeroS):
    mesh = plsc.VectorSubcoreMesh(core_axis_name="c", subcore_axis_name="s")
    fn = pl.kernel(
        _sc_edge_body,
        mesh=mesh,
        compiler_params=pltpu.CompilerParams(use_tc_tiling_on_sc=False),
        out_type=jax.ShapeDtypeStruct((NSC * R, SW), jnp.float32),
        scratch_types=[
            pltpu.VMEM((4, 2, CHUNK), jnp.int32),
            pltpu.VMEM((2, CHUNK, CH), jnp.float32),
            pltpu.VMEM((2, CHUNK, CH), jnp.float32),
            pltpu.VMEM((2, CHUNK, D), jnp.float32),
            pltpu.VMEM((2, CHUNK, SW), jnp.float32),
            pltpu.VMEM_SHARED((R, SW), jnp.float32),
            pltpu.SemaphoreType.DMA,
            pltpu.SemaphoreType.DMA,
            pltpu.SemaphoreType.DMA((2,)),
        ],
    )
    return fn(edges, A, Dt, H, zeroS)


def _expand_att(a):
    af = a.reshape(-1)
    heads = jnp.arange(D, dtype=jnp.int32) // CH
    cols = jnp.arange(CH, dtype=jnp.int32)
    m = (cols[None, :] % HEADS) == heads[:, None]
    return jnp.where(m, af[:, None], 0.0).astype(jnp.float32)


def kernel(x, edge_index, W0, a_s0, a_d0, b0, W1, a_s1, a_d1, b1, Wf, bf):
    f32 = jnp.float32
    pad_e = EPAD - edge_index.shape[1]
    pad_ids = N + (jnp.arange(pad_e, dtype=jnp.int32) % (R - N))
    edges = jnp.concatenate(
        [edge_index, jnp.tile(pad_ids, (2, 1))], axis=1)

    As0 = _expand_att(a_s0)
    Ad0 = _expand_att(a_d0)
    As1 = _expand_att(a_s1)
    Ad1 = _expand_att(a_d1)
    P = ((jnp.arange(CH)[:, None] == (jnp.arange(D)[None, :] // CH))
         .astype(f32))
    b0_row = b0.reshape(1, D).astype(f32)
    b1_row = b1.reshape(1, D).astype(f32)
    Wf_pad = jnp.zeros((D, D), f32).at[:, :OUT].set(Wf)
    bf_row = jnp.zeros((1, D), f32).at[0, :OUT].set(bf)
    zeroS = jnp.zeros((ROWS_PER_TILE, SW), f32)

    h0, a0, d0, ws0 = _tc_pre(x, W0, As0, Ad0)
    S0 = _sc_edge_pass(edges, a0, d0, h0, zeroS)
    h1, g1, a1, d1, ws1 = _tc_mid(S0, h0, ws0, P, b0_row, W1, As1, Ad1)
    S1 = _sc_edge_pass(edges, a1, d1, g1, zeroS)
    return _tc_fin(S1, g1, h1, ws1, P, b1_row, Wf_pad, bf_row)
